# baseline (speedup 1.0000x reference)
import jax
import jax.numpy as jnp
from jax import lax
from jax.experimental import pallas as pl
from jax.experimental.pallas import tpu as pltpu
from jax.experimental.pallas import tpu_sc as plsc

_N = 10000
_E = 320000
_ND = 128
_ED = 16
_H = 64
_W = 128

_CW = 80
_G = 5
_GE = _G * _CW
_EPT = _E // 32
_EPS = _E // 16
_NHALF = 5000
_NACC = 5008

_BE = 2560
_BN = 2000

_f32 = jnp.float32


def _mesh():
    return plsc.VectorSubcoreMesh(core_axis_name="c", subcore_axis_name="s")


def _pad_rows(w):
    return jnp.concatenate([w, jnp.zeros_like(w)], axis=0)


def _dup_cols(w):
    return jnp.concatenate([w, w], axis=1)


def _gather_body(h_hbm, s_hbm, d_hbm, hsd_hbm,
                 idxs, idxd, rows_s, rows_d, sem_s, sem_d):
    cid = lax.axis_index("c")
    sid = lax.axis_index("s")
    wid = sid * 2 + cid

    def step(i, carry):
        ebase = wid * _EPT + i * _GE
        pltpu.sync_copy(s_hbm.at[pl.ds(ebase, _GE)], idxs)
        pltpu.sync_copy(d_hbm.at[pl.ds(ebase, _GE)], idxd)
        cps = [pltpu.async_copy(h_hbm.at[idxs.at[pl.ds(j * _CW, _CW)]],
                                rows_s.at[pl.ds(j * _CW, _CW)], sem_s)
               for j in range(_G)]
        cpd = [pltpu.async_copy(h_hbm.at[idxd.at[pl.ds(j * _CW, _CW)]],
                                rows_d.at[pl.ds(j * _CW, _CW)], sem_d)
               for j in range(_G)]
        for c in cps:
            c.wait()
        for c in cpd:
            c.wait()

        def splice(r, carry2):
            for c in range(4):
                o = _H + c * 16
                rows_s[r, pl.ds(o, 16)] = rows_d[r, pl.ds(o, 16)]
            return carry2

        lax.fori_loop(0, _GE, splice, 0)
        pltpu.sync_copy(rows_s, hsd_hbm.at[pl.ds(ebase, _GE)])
        return carry

    lax.fori_loop(0, _EPT // _GE, step, 0)


def _gather2(h, src, dst):
    f = pl.kernel(
        _gather_body,
        out_type=jax.ShapeDtypeStruct((_E, _W), _f32),
        mesh=_mesh(),
        compiler_params=pltpu.CompilerParams(use_tc_tiling_on_sc=True),
        scratch_types=[
            pltpu.VMEM((_GE,), jnp.int32),
            pltpu.VMEM((_GE,), jnp.int32),
            pltpu.VMEM((_GE, _W), _f32),
            pltpu.VMEM((_GE, _W), _f32),
            pltpu.SemaphoreType.DMA,
            pltpu.SemaphoreType.DMA,
        ],
    )
    return f(h, src, dst)


def _scatter_body(e_hbm, d_hbm, z_hbm, agg_hbm,
                  acc, idxd, rows, c0, c1, c2, c3, c4, sem):
    cid = lax.axis_index("c")
    sid = lax.axis_index("s")
    cbase = cid * _NHALF
    chunks = (c0, c1, c2, c3, c4)

    @pl.when(sid == 0)
    def _():
        pltpu.sync_copy(z_hbm, acc)

    plsc.subcore_barrier()

    def step(i, carry):
        ebase = sid * _EPS + i * _GE
        pltpu.sync_copy(d_hbm.at[pl.ds(ebase, _GE)], idxd)
        pltpu.sync_copy(e_hbm.at[pl.ds(ebase, _GE)], rows)
        for j in range(_G):
            for v in range(_CW // 16):
                o = j * _CW + v * 16
                t = idxd[pl.ds(o, 16)] - cbase
                oob = (t < 0) | (t >= _NHALF)
                chunks[j][pl.ds(v * 16, 16)] = jnp.where(oob, _NHALF, t)
        cps = [pltpu.async_copy(rows.at[pl.ds(j * _CW, _CW)],
                                acc.at[chunks[j]], sem, add=True)
               for j in range(_G)]
        for c in cps:
            c.wait()
        return carry

    lax.fori_loop(0, _EPS // _GE, step, 0)
    plsc.subcore_barrier()

    @pl.when(sid == 0)
    def _():
        pltpu.sync_copy(acc.at[pl.ds(0, _NHALF)],
                        agg_hbm.at[pl.ds(cbase, _NHALF)])


def _scatter(e, dst, zeros):
    f = pl.kernel(
        _scatter_body,
        out_type=jax.ShapeDtypeStruct((_N, _W), _f32),
        mesh=_mesh(),
        compiler_params=pltpu.CompilerParams(use_tc_tiling_on_sc=True),
        scratch_types=[
            pltpu.VMEM_SHARED((_NACC, _W), _f32),
            pltpu.VMEM((_GE,), jnp.int32),
            pltpu.VMEM((_GE, _W), _f32),
            pltpu.VMEM((_CW,), jnp.int32),
            pltpu.VMEM((_CW,), jnp.int32),
            pltpu.VMEM((_CW,), jnp.int32),
            pltpu.VMEM((_CW,), jnp.int32),
            pltpu.VMEM((_CW,), jnp.int32),
            pltpu.SemaphoreType.DMA,
        ],
    )
    return f(e, dst, zeros)


def _dot(a, b):
    return jnp.dot(a, b, preferred_element_type=_f32,
                   precision=lax.Precision.HIGHEST)


def _node_enc_body(x_ref, w1, b1, w2, b2, o_ref):
    u = jnp.maximum(_dot(x_ref[...], w1[...]) + b1[...], 0.0)
    o_ref[...] = _dot(u, w2[...]) + b2[...]


def _node_enc(x, p):
    (W1, b1), (W2, b2) = p
    grid = _N // _BN
    cst = lambda i: (0, 0)
    return pl.pallas_call(
        _node_enc_body,
        grid=(grid,),
        in_specs=[pl.BlockSpec((_BN, _ND), lambda i: (i, 0)),
                  pl.BlockSpec((_ND, _H), cst),
                  pl.BlockSpec((1, _H), cst),
                  pl.BlockSpec((_H, _W), cst),
                  pl.BlockSpec((1, _W), cst)],
        out_specs=pl.BlockSpec((_BN, _W), lambda i: (i, 0)),
        out_shape=jax.ShapeDtypeStruct((_N, _W), _f32),
    )(x, W1, b1.reshape(1, -1), _dup_cols(W2), _dup_cols(b2.reshape(1, -1)))


def _edge_round1_body(eaT_ref, hsd_ref, e1, be1, e2, be2,
                      w1ab, w1c, b1, w2, b2, o_ref):
    e0 = lax.dot_general(eaT_ref[...], e1[...], (((0,), (0,)), ((), ())),
                         preferred_element_type=_f32,
                         precision=lax.Precision.HIGHEST)
    e0 = jnp.maximum(e0 + be1[...], 0.0)
    e0 = _dot(e0, e2[...]) + be2[...]
    u = _dot(hsd_ref[...], w1ab[...])
    u = u + _dot(e0, w1c[...])
    u = jnp.maximum(u + b1[...], 0.0)
    o_ref[...] = _dot(u, w2[...]) + b2[...]


def _edge_round1(edge_attr_T, hsd, enc_p, upd_p):
    (E1, be1), (E2, be2) = enc_p
    (W1, b1), (W2, b2) = upd_p
    grid = _E // _BE
    cst = lambda i: (0, 0)
    return pl.pallas_call(
        _edge_round1_body,
        grid=(grid,),
        in_specs=[pl.BlockSpec((_ED, _BE), lambda i: (0, i)),
                  pl.BlockSpec((_BE, _W), lambda i: (i, 0)),
                  pl.BlockSpec((_ED, _H), cst),
                  pl.BlockSpec((1, _H), cst),
                  pl.BlockSpec((_H, _H), cst),
                  pl.BlockSpec((1, _H), cst),
                  pl.BlockSpec((_W, _W), cst),
                  pl.BlockSpec((_H, _W), cst),
                  pl.BlockSpec((1, _W), cst),
                  pl.BlockSpec((_W, _W), cst),
                  pl.BlockSpec((1, _W), cst)],
        out_specs=pl.BlockSpec((_BE, _W), lambda i: (i, 0)),
        out_shape=jax.ShapeDtypeStruct((_E, _W), _f32),
    )(edge_attr_T, hsd, E1, be1.reshape(1, -1), E2, be2.reshape(1, -1),
      W1[:2 * _H], W1[2 * _H:],
      b1.reshape(1, -1), _dup_cols(W2), _dup_cols(b2.reshape(1, -1)))


def _edge_upd_body(e_ref, hsd_ref, w1ab, w1c, b1, w2, b2, o_ref):
    u = _dot(hsd_ref[...], w1ab[...])
    u = u + _dot(e_ref[...], w1c[...])
    u = jnp.maximum(u + b1[...], 0.0)
    o_ref[...] = _dot(u, w2[...]) + b2[...]


def _edge_upd(e, hsd, upd_p):
    (W1, b1), (W2, b2) = upd_p
    grid = _E // _BE
    cst = lambda i: (0, 0)
    return pl.pallas_call(
        _edge_upd_body,
        grid=(grid,),
        in_specs=[pl.BlockSpec((_BE, _W), lambda i: (i, 0)),
                  pl.BlockSpec((_BE, _W), lambda i: (i, 0)),
                  pl.BlockSpec((_W, _W), cst),
                  pl.BlockSpec((_W, _W), cst),
                  pl.BlockSpec((1, _W), cst),
                  pl.BlockSpec((_W, _W), cst),
                  pl.BlockSpec((1, _W), cst)],
        out_specs=pl.BlockSpec((_BE, _W), lambda i: (i, 0)),
        out_shape=jax.ShapeDtypeStruct((_E, _W), _f32),
    )(e, hsd, W1[:2 * _H], _pad_rows(W1[2 * _H:]), b1.reshape(1, -1),
      _dup_cols(W2), _dup_cols(b2.reshape(1, -1)))


def _node_upd_body(h_ref, a_ref, v1a, v1b, b1, v2, b2, o_ref):
    u = _dot(h_ref[...], v1a[...]) + _dot(a_ref[...], v1b[...])
    u = jnp.maximum(u + b1[...], 0.0)
    o_ref[...] = _dot(u, v2[...]) + b2[...]


def _node_upd(h, agg, upd_p):
    (V1, b1), (V2, b2) = upd_p
    grid = _N // _BN
    cst = lambda i: (0, 0)
    return pl.pallas_call(
        _node_upd_body,
        grid=(grid,),
        in_specs=[pl.BlockSpec((_BN, _W), lambda i: (i, 0)),
                  pl.BlockSpec((_BN, _W), lambda i: (i, 0)),
                  pl.BlockSpec((_W, _W), cst),
                  pl.BlockSpec((_W, _W), cst),
                  pl.BlockSpec((1, _W), cst),
                  pl.BlockSpec((_W, _W), cst),
                  pl.BlockSpec((1, _W), cst)],
        out_specs=pl.BlockSpec((_BN, _W), lambda i: (i, 0)),
        out_shape=jax.ShapeDtypeStruct((_N, _W), _f32),
    )(h, agg, _pad_rows(V1[:_H]), _pad_rows(V1[_H:]), b1.reshape(1, -1),
      _dup_cols(V2), _dup_cols(b2.reshape(1, -1)))


def _head_body(e_ref, hsd_ref, w1ab, w1c, b1, w2, b2, w3, b3, o_ref):
    u = _dot(hsd_ref[...], w1ab[...])
    u = u + _dot(e_ref[...], w1c[...])
    u = jnp.maximum(u + b1[...], 0.0)
    u = jnp.maximum(_dot(u, w2[...]) + b2[...], 0.0)
    o_ref[...] = _dot(u, w3[...]) + b3[...]


def _head(e, hsd, head_p):
    (W1, b1), (W2, b2), (W3, b3) = head_p
    grid = _E // _BE
    cst = lambda i: (0, 0)
    out = pl.pallas_call(
        _head_body,
        grid=(grid,),
        in_specs=[pl.BlockSpec((_BE, _W), lambda i: (i, 0)),
                  pl.BlockSpec((_BE, _W), lambda i: (i, 0)),
                  pl.BlockSpec((_W, _W), cst),
                  pl.BlockSpec((_W, _W), cst),
                  pl.BlockSpec((1, _W), cst),
                  pl.BlockSpec((_W, _H), cst),
                  pl.BlockSpec((1, _H), cst),
                  pl.BlockSpec((_H, 1), cst),
                  pl.BlockSpec((1, 1), cst)],
        out_specs=pl.BlockSpec((_BE, 1), lambda i: (i, 0)),
        out_shape=jax.ShapeDtypeStruct((_E, 1), _f32),
    )(e, hsd, W1[:2 * _H], _pad_rows(W1[2 * _H:]), b1.reshape(1, -1),
      W2, b2.reshape(1, -1), W3, b3.reshape(1, 1))
    return out.reshape(_E)


def kernel(x, edge_attr, params, edge_index):
    src = edge_index[0]
    dst = edge_index[1]
    zeros = jnp.zeros((_NACC, _W), _f32)

    h = _node_enc(x, params['node_enc'])

    hsd = _gather2(h, src, dst)
    e = _edge_round1(edge_attr.T, hsd, params['edge_enc'],
                     params['edge_upd'][0])
    agg = _scatter(e, dst, zeros)
    h = _node_upd(h, agg, params['node_upd'][0])

    for k in (1, 2):
        hsd = _gather2(h, src, dst)
        e = _edge_upd(e, hsd, params['edge_upd'][k])
        agg = _scatter(e, dst, zeros)
        h = _node_upd(h, agg, params['node_upd'][k])

    hsd = _gather2(h, src, dst)
    return _head(e, hsd, params['edge_head'])


# speedup vs baseline: 1.7865x; 1.7865x over previous
import jax
import jax.numpy as jnp
from jax import lax
from jax.experimental import pallas as pl
from jax.experimental.pallas import tpu as pltpu
from jax.experimental.pallas import tpu_sc as plsc

_N = 10000
_E = 320000
_ND = 128
_ED = 16
_H = 64
_W = 128

_CW = 80
_G = 5
_GE = _G * _CW
_EPT = _E // 32
_EPS = _E // 16
_NHALF = 5000
_NACC = 5008

_BE = 2560
_BN = 2000

_f32 = jnp.float32


def _mesh():
    return plsc.VectorSubcoreMesh(core_axis_name="c", subcore_axis_name="s")


def _pad_rows(w):
    return jnp.concatenate([w, jnp.zeros_like(w)], axis=0)


def _dup_cols(w):
    return jnp.concatenate([w, w], axis=1)


def _gather_body(h_hbm, s_hbm, d_hbm, hsd_hbm,
                 idxs, idxd, rows_s, rows_d, sem_s, sem_d):
    cid = lax.axis_index("c")
    sid = lax.axis_index("s")
    wid = sid * 2 + cid

    def step(i, carry):
        ebase = wid * _EPT + i * _GE
        pltpu.sync_copy(s_hbm.at[pl.ds(ebase, _GE)], idxs)
        pltpu.sync_copy(d_hbm.at[pl.ds(ebase, _GE)], idxd)
        cps = [pltpu.async_copy(h_hbm.at[idxs.at[pl.ds(j * _CW, _CW)]],
                                rows_s.at[pl.ds(j * _CW, _CW)], sem_s)
               for j in range(_G)]
        cpd = [pltpu.async_copy(h_hbm.at[idxd.at[pl.ds(j * _CW, _CW)]],
                                rows_d.at[pl.ds(j * _CW, _CW)], sem_d)
               for j in range(_G)]
        for c in cps:
            c.wait()
        for c in cpd:
            c.wait()

        def splice(r, carry2):
            for c in range(4):
                o = _H + c * 16
                rows_s[r, pl.ds(o, 16)] = rows_d[r, pl.ds(o, 16)]
            return carry2

        lax.fori_loop(0, _GE, splice, 0)
        pltpu.sync_copy(rows_s, hsd_hbm.at[pl.ds(ebase, _GE)])
        return carry

    lax.fori_loop(0, _EPT // _GE, step, 0)


def _gather2(h, src, dst):
    f = pl.kernel(
        _gather_body,
        out_type=jax.ShapeDtypeStruct((_E, _W), _f32),
        mesh=_mesh(),
        compiler_params=pltpu.CompilerParams(use_tc_tiling_on_sc=True),
        scratch_types=[
            pltpu.VMEM((_GE,), jnp.int32),
            pltpu.VMEM((_GE,), jnp.int32),
            pltpu.VMEM((_GE, _W), _f32),
            pltpu.VMEM((_GE, _W), _f32),
            pltpu.SemaphoreType.DMA,
            pltpu.SemaphoreType.DMA,
        ],
    )
    return f(h, src, dst)


def _scatter_body(e_hbm, d_hbm, z_hbm, agg_hbm,
                  acc, idxd, rows, c0, c1, c2, c3, c4, sem):
    cid = lax.axis_index("c")
    sid = lax.axis_index("s")
    cbase = cid * _NHALF
    chunks = (c0, c1, c2, c3, c4)

    @pl.when(sid == 0)
    def _():
        pltpu.sync_copy(z_hbm, acc)

    plsc.subcore_barrier()

    def step(i, carry):
        ebase = sid * _EPS + i * _GE
        pltpu.sync_copy(d_hbm.at[pl.ds(ebase, _GE)], idxd)
        pltpu.sync_copy(e_hbm.at[pl.ds(ebase, _GE)], rows)
        for j in range(_G):
            for v in range(_CW // 16):
                o = j * _CW + v * 16
                t = idxd[pl.ds(o, 16)] - cbase
                oob = (t < 0) | (t >= _NHALF)
                chunks[j][pl.ds(v * 16, 16)] = jnp.where(oob, _NHALF, t)
        cps = [pltpu.async_copy(rows.at[pl.ds(j * _CW, _CW)],
                                acc.at[chunks[j]], sem, add=True)
               for j in range(_G)]
        for c in cps:
            c.wait()
        return carry

    lax.fori_loop(0, _EPS // _GE, step, 0)
    plsc.subcore_barrier()

    @pl.when(sid == 0)
    def _():
        pltpu.sync_copy(acc.at[pl.ds(0, _NHALF)],
                        agg_hbm.at[pl.ds(cbase, _NHALF)])


def _scatter(e, dst, zeros):
    f = pl.kernel(
        _scatter_body,
        out_type=jax.ShapeDtypeStruct((_N, _W), _f32),
        mesh=_mesh(),
        compiler_params=pltpu.CompilerParams(use_tc_tiling_on_sc=True),
        scratch_types=[
            pltpu.VMEM_SHARED((_NACC, _W), _f32),
            pltpu.VMEM((_GE,), jnp.int32),
            pltpu.VMEM((_GE, _W), _f32),
            pltpu.VMEM((_CW,), jnp.int32),
            pltpu.VMEM((_CW,), jnp.int32),
            pltpu.VMEM((_CW,), jnp.int32),
            pltpu.VMEM((_CW,), jnp.int32),
            pltpu.VMEM((_CW,), jnp.int32),
            pltpu.SemaphoreType.DMA,
        ],
    )
    return f(e, dst, zeros)


def _dot(a, b):
    return jnp.dot(a, b, preferred_element_type=_f32)


def _node_enc_body(x_ref, w1, b1, w2, b2, o_ref):
    u = jnp.maximum(_dot(x_ref[...], w1[...]) + b1[...], 0.0)
    o_ref[...] = _dot(u, w2[...]) + b2[...]


def _node_enc(x, p):
    (W1, b1), (W2, b2) = p
    grid = _N // _BN
    cst = lambda i: (0, 0)
    return pl.pallas_call(
        _node_enc_body,
        grid=(grid,),
        in_specs=[pl.BlockSpec((_BN, _ND), lambda i: (i, 0)),
                  pl.BlockSpec((_ND, _H), cst),
                  pl.BlockSpec((1, _H), cst),
                  pl.BlockSpec((_H, _W), cst),
                  pl.BlockSpec((1, _W), cst)],
        out_specs=pl.BlockSpec((_BN, _W), lambda i: (i, 0)),
        out_shape=jax.ShapeDtypeStruct((_N, _W), _f32),
    )(x, W1, b1.reshape(1, -1), _dup_cols(W2), _dup_cols(b2.reshape(1, -1)))


def _edge_round1_body(eaT_ref, hsd_ref, e1, be1, e2, be2,
                      w1ab, w1c, b1, w2, b2, o_ref):
    e0 = lax.dot_general(eaT_ref[...], e1[...], (((0,), (0,)), ((), ())),
                         preferred_element_type=_f32,
                         precision=lax.Precision.HIGHEST)
    e0 = jnp.maximum(e0 + be1[...], 0.0)
    e0 = _dot(e0, e2[...]) + be2[...]
    u = _dot(hsd_ref[...], w1ab[...])
    u = u + _dot(e0, w1c[...])
    u = jnp.maximum(u + b1[...], 0.0)
    o_ref[...] = _dot(u, w2[...]) + b2[...]


def _edge_round1(edge_attr_T, hsd, enc_p, upd_p):
    (E1, be1), (E2, be2) = enc_p
    (W1, b1), (W2, b2) = upd_p
    grid = _E // _BE
    cst = lambda i: (0, 0)
    return pl.pallas_call(
        _edge_round1_body,
        grid=(grid,),
        in_specs=[pl.BlockSpec((_ED, _BE), lambda i: (0, i)),
                  pl.BlockSpec((_BE, _W), lambda i: (i, 0)),
                  pl.BlockSpec((_ED, _H), cst),
                  pl.BlockSpec((1, _H), cst),
                  pl.BlockSpec((_H, _H), cst),
                  pl.BlockSpec((1, _H), cst),
                  pl.BlockSpec((_W, _W), cst),
                  pl.BlockSpec((_H, _W), cst),
                  pl.BlockSpec((1, _W), cst),
                  pl.BlockSpec((_W, _W), cst),
                  pl.BlockSpec((1, _W), cst)],
        out_specs=pl.BlockSpec((_BE, _W), lambda i: (i, 0)),
        out_shape=jax.ShapeDtypeStruct((_E, _W), _f32),
    )(edge_attr_T, hsd, E1, be1.reshape(1, -1), E2, be2.reshape(1, -1),
      W1[:2 * _H], W1[2 * _H:],
      b1.reshape(1, -1), _dup_cols(W2), _dup_cols(b2.reshape(1, -1)))


def _edge_upd_body(e_ref, hsd_ref, w1ab, w1c, b1, w2, b2, o_ref):
    u = _dot(hsd_ref[...], w1ab[...])
    u = u + _dot(e_ref[...], w1c[...])
    u = jnp.maximum(u + b1[...], 0.0)
    o_ref[...] = _dot(u, w2[...]) + b2[...]


def _edge_upd(e, hsd, upd_p):
    (W1, b1), (W2, b2) = upd_p
    grid = _E // _BE
    cst = lambda i: (0, 0)
    return pl.pallas_call(
        _edge_upd_body,
        grid=(grid,),
        in_specs=[pl.BlockSpec((_BE, _W), lambda i: (i, 0)),
                  pl.BlockSpec((_BE, _W), lambda i: (i, 0)),
                  pl.BlockSpec((_W, _W), cst),
                  pl.BlockSpec((_W, _W), cst),
                  pl.BlockSpec((1, _W), cst),
                  pl.BlockSpec((_W, _W), cst),
                  pl.BlockSpec((1, _W), cst)],
        out_specs=pl.BlockSpec((_BE, _W), lambda i: (i, 0)),
        out_shape=jax.ShapeDtypeStruct((_E, _W), _f32),
    )(e, hsd, W1[:2 * _H], _pad_rows(W1[2 * _H:]), b1.reshape(1, -1),
      _dup_cols(W2), _dup_cols(b2.reshape(1, -1)))


def _node_upd_body(h_ref, a_ref, v1a, v1b, b1, v2, b2, o_ref):
    u = _dot(h_ref[...], v1a[...]) + _dot(a_ref[...], v1b[...])
    u = jnp.maximum(u + b1[...], 0.0)
    o_ref[...] = _dot(u, v2[...]) + b2[...]


def _node_upd(h, agg, upd_p):
    (V1, b1), (V2, b2) = upd_p
    grid = _N // _BN
    cst = lambda i: (0, 0)
    return pl.pallas_call(
        _node_upd_body,
        grid=(grid,),
        in_specs=[pl.BlockSpec((_BN, _W), lambda i: (i, 0)),
                  pl.BlockSpec((_BN, _W), lambda i: (i, 0)),
                  pl.BlockSpec((_W, _W), cst),
                  pl.BlockSpec((_W, _W), cst),
                  pl.BlockSpec((1, _W), cst),
                  pl.BlockSpec((_W, _W), cst),
                  pl.BlockSpec((1, _W), cst)],
        out_specs=pl.BlockSpec((_BN, _W), lambda i: (i, 0)),
        out_shape=jax.ShapeDtypeStruct((_N, _W), _f32),
    )(h, agg, _pad_rows(V1[:_H]), _pad_rows(V1[_H:]), b1.reshape(1, -1),
      _dup_cols(V2), _dup_cols(b2.reshape(1, -1)))


def _head_body(e_ref, hsd_ref, w1ab, w1c, b1, w2, b2, w3, b3, o_ref):
    u = _dot(hsd_ref[...], w1ab[...])
    u = u + _dot(e_ref[...], w1c[...])
    u = jnp.maximum(u + b1[...], 0.0)
    u = jnp.maximum(_dot(u, w2[...]) + b2[...], 0.0)
    o_ref[...] = _dot(u, w3[...]) + b3[...]


def _head(e, hsd, head_p):
    (W1, b1), (W2, b2), (W3, b3) = head_p
    grid = _E // _BE
    cst = lambda i: (0, 0)
    out = pl.pallas_call(
        _head_body,
        grid=(grid,),
        in_specs=[pl.BlockSpec((_BE, _W), lambda i: (i, 0)),
                  pl.BlockSpec((_BE, _W), lambda i: (i, 0)),
                  pl.BlockSpec((_W, _W), cst),
                  pl.BlockSpec((_W, _W), cst),
                  pl.BlockSpec((1, _W), cst),
                  pl.BlockSpec((_W, _H), cst),
                  pl.BlockSpec((1, _H), cst),
                  pl.BlockSpec((_H, 1), cst),
                  pl.BlockSpec((1, 1), cst)],
        out_specs=pl.BlockSpec((_BE, 1), lambda i: (i, 0)),
        out_shape=jax.ShapeDtypeStruct((_E, 1), _f32),
    )(e, hsd, W1[:2 * _H], _pad_rows(W1[2 * _H:]), b1.reshape(1, -1),
      W2, b2.reshape(1, -1), W3, b3.reshape(1, 1))
    return out.reshape(_E)


def kernel(x, edge_attr, params, edge_index):
    src = edge_index[0]
    dst = edge_index[1]
    zeros = jnp.zeros((_NACC, _W), _f32)

    h = _node_enc(x, params['node_enc'])

    hsd = _gather2(h, src, dst)
    e = _edge_round1(edge_attr.T, hsd, params['edge_enc'],
                     params['edge_upd'][0])
    agg = _scatter(e, dst, zeros)
    h = _node_upd(h, agg, params['node_upd'][0])

    for k in (1, 2):
        hsd = _gather2(h, src, dst)
        e = _edge_upd(e, hsd, params['edge_upd'][k])
        agg = _scatter(e, dst, zeros)
        h = _node_upd(h, agg, params['node_upd'][k])

    hsd = _gather2(h, src, dst)
    return _head(e, hsd, params['edge_head'])


# speedup vs baseline: 1.8636x; 1.0432x over previous
import jax
import jax.numpy as jnp
from jax import lax
from jax.experimental import pallas as pl
from jax.experimental.pallas import tpu as pltpu
from jax.experimental.pallas import tpu_sc as plsc

_N = 10000
_E = 320000
_ND = 128
_ED = 16
_H = 64
_W = 128

_CW = 80
_G = 5
_GE = _G * _CW
_EPT = _E // 32
_EPS = _E // 16
_NHALF = 5000
_NACC = 5008

_BE = 2560
_BN = 2000

_f32 = jnp.float32


def _mesh():
    return plsc.VectorSubcoreMesh(core_axis_name="c", subcore_axis_name="s")


def _pad_rows(w):
    return jnp.concatenate([w, jnp.zeros_like(w)], axis=0)


def _dup_cols(w):
    return jnp.concatenate([w, w], axis=1)


def _gather_body(h_hbm, s_hbm, d_hbm, hsd_hbm,
                 idxs, idxd, rows_s, rows_d, sem_s, sem_d):
    cid = lax.axis_index("c")
    sid = lax.axis_index("s")
    wid = sid * 2 + cid

    def step(i, carry):
        ebase = wid * _EPT + i * _GE
        pltpu.sync_copy(s_hbm.at[pl.ds(ebase, _GE)], idxs)
        pltpu.sync_copy(d_hbm.at[pl.ds(ebase, _GE)], idxd)
        cps = [pltpu.async_copy(h_hbm.at[idxs.at[pl.ds(j * _CW, _CW)]],
                                rows_s.at[pl.ds(j * _CW, _CW)], sem_s)
               for j in range(_G)]
        cpd = [pltpu.async_copy(h_hbm.at[idxd.at[pl.ds(j * _CW, _CW)]],
                                rows_d.at[pl.ds(j * _CW, _CW)], sem_d)
               for j in range(_G)]
        for c in cps:
            c.wait()
        for c in cpd:
            c.wait()

        def splice(r, carry2):
            for c in range(4):
                o = _H + c * 16
                rows_s[r, pl.ds(o, 16)] = rows_d[r, pl.ds(o, 16)]
            return carry2

        lax.fori_loop(0, _GE, splice, 0)
        pltpu.sync_copy(rows_s, hsd_hbm.at[pl.ds(ebase, _GE)])
        return carry

    lax.fori_loop(0, _EPT // _GE, step, 0)


def _gather2(h, src, dst):
    f = pl.kernel(
        _gather_body,
        out_type=jax.ShapeDtypeStruct((_E, _W), _f32),
        mesh=_mesh(),
        compiler_params=pltpu.CompilerParams(use_tc_tiling_on_sc=True),
        scratch_types=[
            pltpu.VMEM((_GE,), jnp.int32),
            pltpu.VMEM((_GE,), jnp.int32),
            pltpu.VMEM((_GE, _W), _f32),
            pltpu.VMEM((_GE, _W), _f32),
            pltpu.SemaphoreType.DMA,
            pltpu.SemaphoreType.DMA,
        ],
    )
    return f(h, src, dst)


def _scatter_body(e_hbm, d_hbm, z_hbm, agg_hbm,
                  acc, idxd, rows, c0, c1, c2, c3, c4, sem):
    cid = lax.axis_index("c")
    sid = lax.axis_index("s")
    cbase = cid * _NHALF
    chunks = (c0, c1, c2, c3, c4)

    @pl.when(sid == 0)
    def _():
        pltpu.sync_copy(z_hbm, acc)

    plsc.subcore_barrier()

    def step(i, carry):
        ebase = sid * _EPS + i * _GE
        pltpu.sync_copy(d_hbm.at[pl.ds(ebase, _GE)], idxd)
        pltpu.sync_copy(e_hbm.at[pl.ds(ebase, _GE)], rows)
        for j in range(_G):
            for v in range(_CW // 16):
                o = j * _CW + v * 16
                t = idxd[pl.ds(o, 16)] - cbase
                oob = (t < 0) | (t >= _NHALF)
                chunks[j][pl.ds(v * 16, 16)] = jnp.where(oob, _NHALF, t)
        cps = [pltpu.async_copy(rows.at[pl.ds(j * _CW, _CW)],
                                acc.at[chunks[j]], sem, add=True)
               for j in range(_G)]
        for c in cps:
            c.wait()
        return carry

    lax.fori_loop(0, _EPS // _GE, step, 0)
    plsc.subcore_barrier()

    @pl.when(sid == 0)
    def _():
        pltpu.sync_copy(acc.at[pl.ds(0, _NHALF)],
                        agg_hbm.at[pl.ds(cbase, _NHALF)])


def _scatter(e, dst, zeros):
    f = pl.kernel(
        _scatter_body,
        out_type=jax.ShapeDtypeStruct((_N, _W), _f32),
        mesh=_mesh(),
        compiler_params=pltpu.CompilerParams(use_tc_tiling_on_sc=True),
        scratch_types=[
            pltpu.VMEM_SHARED((_NACC, _W), _f32),
            pltpu.VMEM((_GE,), jnp.int32),
            pltpu.VMEM((_GE, _W), _f32),
            pltpu.VMEM((_CW,), jnp.int32),
            pltpu.VMEM((_CW,), jnp.int32),
            pltpu.VMEM((_CW,), jnp.int32),
            pltpu.VMEM((_CW,), jnp.int32),
            pltpu.VMEM((_CW,), jnp.int32),
            pltpu.SemaphoreType.DMA,
        ],
    )
    return f(e, dst, zeros)


def _dot(a, b):
    return jnp.dot(a, b, preferred_element_type=_f32)


def _node_enc_body(x_ref, w1, b1, w2, b2, o_ref):
    u = jnp.maximum(_dot(x_ref[...], w1[...]) + b1[...], 0.0)
    o_ref[...] = _dot(u, w2[...]) + b2[...]


def _node_enc(x, p):
    (W1, b1), (W2, b2) = p
    grid = _N // _BN
    cst = lambda i: (0, 0)
    return pl.pallas_call(
        _node_enc_body,
        grid=(grid,),
        in_specs=[pl.BlockSpec((_BN, _ND), lambda i: (i, 0)),
                  pl.BlockSpec((_ND, _H), cst),
                  pl.BlockSpec((1, _H), cst),
                  pl.BlockSpec((_H, _W), cst),
                  pl.BlockSpec((1, _W), cst)],
        out_specs=pl.BlockSpec((_BN, _W), lambda i: (i, 0)),
        out_shape=jax.ShapeDtypeStruct((_N, _W), _f32),
    )(x, W1, b1.reshape(1, -1), _dup_cols(W2), _dup_cols(b2.reshape(1, -1)))


def _edge_round1_body(ea_ref, hsd_ref, e1, be1, e2, be2,
                      w1ab, w1c, b1, w2, b2, o_ref):
    e0 = jnp.maximum(_dot(ea_ref[...], e1[...]) + be1[...], 0.0)
    e0 = _dot(e0, e2[...]) + be2[...]
    u = _dot(hsd_ref[...], w1ab[...])
    u = u + _dot(e0, w1c[...])
    u = jnp.maximum(u + b1[...], 0.0)
    o_ref[...] = _dot(u, w2[...]) + b2[...]


def _edge_round1(edge_attr, hsd, enc_p, upd_p):
    (E1, be1), (E2, be2) = enc_p
    (W1, b1), (W2, b2) = upd_p
    grid = _E // _BE
    cst = lambda i: (0, 0)
    return pl.pallas_call(
        _edge_round1_body,
        grid=(grid,),
        in_specs=[pl.BlockSpec((_BE, _ED), lambda i: (i, 0)),
                  pl.BlockSpec((_BE, _W), lambda i: (i, 0)),
                  pl.BlockSpec((_ED, _H), cst),
                  pl.BlockSpec((1, _H), cst),
                  pl.BlockSpec((_H, _H), cst),
                  pl.BlockSpec((1, _H), cst),
                  pl.BlockSpec((_W, _W), cst),
                  pl.BlockSpec((_H, _W), cst),
                  pl.BlockSpec((1, _W), cst),
                  pl.BlockSpec((_W, _W), cst),
                  pl.BlockSpec((1, _W), cst)],
        out_specs=pl.BlockSpec((_BE, _W), lambda i: (i, 0)),
        out_shape=jax.ShapeDtypeStruct((_E, _W), _f32),
    )(edge_attr, hsd, E1, be1.reshape(1, -1), E2, be2.reshape(1, -1),
      W1[:2 * _H], W1[2 * _H:],
      b1.reshape(1, -1), _dup_cols(W2), _dup_cols(b2.reshape(1, -1)))


def _edge_upd_body(e_ref, hsd_ref, w1ab, w1c, b1, w2, b2, o_ref):
    u = _dot(hsd_ref[...], w1ab[...])
    u = u + _dot(e_ref[...], w1c[...])
    u = jnp.maximum(u + b1[...], 0.0)
    o_ref[...] = _dot(u, w2[...]) + b2[...]


def _edge_upd(e, hsd, upd_p):
    (W1, b1), (W2, b2) = upd_p
    grid = _E // _BE
    cst = lambda i: (0, 0)
    return pl.pallas_call(
        _edge_upd_body,
        grid=(grid,),
        in_specs=[pl.BlockSpec((_BE, _W), lambda i: (i, 0)),
                  pl.BlockSpec((_BE, _W), lambda i: (i, 0)),
                  pl.BlockSpec((_W, _W), cst),
                  pl.BlockSpec((_W, _W), cst),
                  pl.BlockSpec((1, _W), cst),
                  pl.BlockSpec((_W, _W), cst),
                  pl.BlockSpec((1, _W), cst)],
        out_specs=pl.BlockSpec((_BE, _W), lambda i: (i, 0)),
        out_shape=jax.ShapeDtypeStruct((_E, _W), _f32),
    )(e, hsd, W1[:2 * _H], _pad_rows(W1[2 * _H:]), b1.reshape(1, -1),
      _dup_cols(W2), _dup_cols(b2.reshape(1, -1)))


def _node_upd_body(h_ref, a_ref, v1a, v1b, b1, v2, b2, o_ref):
    u = _dot(h_ref[...], v1a[...]) + _dot(a_ref[...], v1b[...])
    u = jnp.maximum(u + b1[...], 0.0)
    o_ref[...] = _dot(u, v2[...]) + b2[...]


def _node_upd(h, agg, upd_p):
    (V1, b1), (V2, b2) = upd_p
    grid = _N // _BN
    cst = lambda i: (0, 0)
    return pl.pallas_call(
        _node_upd_body,
        grid=(grid,),
        in_specs=[pl.BlockSpec((_BN, _W), lambda i: (i, 0)),
                  pl.BlockSpec((_BN, _W), lambda i: (i, 0)),
                  pl.BlockSpec((_W, _W), cst),
                  pl.BlockSpec((_W, _W), cst),
                  pl.BlockSpec((1, _W), cst),
                  pl.BlockSpec((_W, _W), cst),
                  pl.BlockSpec((1, _W), cst)],
        out_specs=pl.BlockSpec((_BN, _W), lambda i: (i, 0)),
        out_shape=jax.ShapeDtypeStruct((_N, _W), _f32),
    )(h, agg, _pad_rows(V1[:_H]), _pad_rows(V1[_H:]), b1.reshape(1, -1),
      _dup_cols(V2), _dup_cols(b2.reshape(1, -1)))


def _head_body(e_ref, hsd_ref, w1ab, w1c, b1, w2, b2, w3, b3, o_ref):
    u = _dot(hsd_ref[...], w1ab[...])
    u = u + _dot(e_ref[...], w1c[...])
    u = jnp.maximum(u + b1[...], 0.0)
    u = jnp.maximum(_dot(u, w2[...]) + b2[...], 0.0)
    o_ref[...] = _dot(u, w3[...]) + b3[...]


def _head(e, hsd, head_p):
    (W1, b1), (W2, b2), (W3, b3) = head_p
    grid = _E // _BE
    cst = lambda i: (0, 0)
    out = pl.pallas_call(
        _head_body,
        grid=(grid,),
        in_specs=[pl.BlockSpec((_BE, _W), lambda i: (i, 0)),
                  pl.BlockSpec((_BE, _W), lambda i: (i, 0)),
                  pl.BlockSpec((_W, _W), cst),
                  pl.BlockSpec((_W, _W), cst),
                  pl.BlockSpec((1, _W), cst),
                  pl.BlockSpec((_W, _H), cst),
                  pl.BlockSpec((1, _H), cst),
                  pl.BlockSpec((_H, 1), cst),
                  pl.BlockSpec((1, 1), cst)],
        out_specs=pl.BlockSpec((_BE, 1), lambda i: (i, 0)),
        out_shape=jax.ShapeDtypeStruct((_E, 1), _f32),
    )(e, hsd, W1[:2 * _H], _pad_rows(W1[2 * _H:]), b1.reshape(1, -1),
      W2, b2.reshape(1, -1), W3, b3.reshape(1, 1))
    return out.reshape(_E)


def kernel(x, edge_attr, params, edge_index):
    src = edge_index[0]
    dst = edge_index[1]
    zeros = jnp.zeros((_NACC, _W), _f32)

    h = _node_enc(x, params['node_enc'])

    hsd = _gather2(h, src, dst)
    e = _edge_round1(edge_attr, hsd, params['edge_enc'],
                     params['edge_upd'][0])
    agg = _scatter(e, dst, zeros)
    h = _node_upd(h, agg, params['node_upd'][0])

    for k in (1, 2):
        hsd = _gather2(h, src, dst)
        e = _edge_upd(e, hsd, params['edge_upd'][k])
        agg = _scatter(e, dst, zeros)
        h = _node_upd(h, agg, params['node_upd'][k])

    hsd = _gather2(h, src, dst)
    return _head(e, hsd, params['edge_head'])


# speedup vs baseline: 1.8988x; 1.0188x over previous
import jax
import jax.numpy as jnp
from jax import lax
from jax.experimental import pallas as pl
from jax.experimental.pallas import tpu as pltpu
from jax.experimental.pallas import tpu_sc as plsc

_N = 10000
_E = 320000
_ND = 128
_ED = 16
_H = 64
_W = 128

_CW = 80
_G = 5
_GE = _G * _CW
_EPT = _E // 32
_EPS = _E // 16
_NHALF = 5000
_NACC = 5008

_BE = 2560
_BN = 2000

_f32 = jnp.float32


def _mesh():
    return plsc.VectorSubcoreMesh(core_axis_name="c", subcore_axis_name="s")


def _pad_rows(w):
    return jnp.concatenate([w, jnp.zeros_like(w)], axis=0)


def _dup_cols(w):
    return jnp.concatenate([w, w], axis=1)


def _gather_body(h_hbm, s_hbm, d_hbm, hsd_hbm,
                 idxs, idxd, rows_s, rows_d, sem_s, sem_d):
    cid = lax.axis_index("c")
    sid = lax.axis_index("s")
    wid = sid * 2 + cid

    def step(i, carry):
        ebase = wid * _EPT + i * _GE
        pltpu.sync_copy(s_hbm.at[pl.ds(ebase, _GE)], idxs)
        pltpu.sync_copy(d_hbm.at[pl.ds(ebase, _GE)], idxd)
        cps = [pltpu.async_copy(h_hbm.at[idxs.at[pl.ds(j * _CW, _CW)]],
                                rows_s.at[pl.ds(j * _CW, _CW)], sem_s)
               for j in range(_G)]
        cpd = [pltpu.async_copy(h_hbm.at[idxd.at[pl.ds(j * _CW, _CW)]],
                                rows_d.at[pl.ds(j * _CW, _CW)], sem_d)
               for j in range(_G)]
        for c in cps:
            c.wait()
        for c in cpd:
            c.wait()

        def splice(r, carry2):
            for c in range(4):
                o = _H + c * 16
                rows_s[r, pl.ds(o, 16)] = rows_d[r, pl.ds(o, 16)]
            return carry2

        lax.fori_loop(0, _GE, splice, 0)
        pltpu.sync_copy(rows_s, hsd_hbm.at[pl.ds(ebase, _GE)])
        return carry

    lax.fori_loop(0, _EPT // _GE, step, 0)


def _gather2(h, src, dst):
    f = pl.kernel(
        _gather_body,
        out_type=jax.ShapeDtypeStruct((_E, _W), _f32),
        mesh=_mesh(),
        compiler_params=pltpu.CompilerParams(use_tc_tiling_on_sc=True),
        scratch_types=[
            pltpu.VMEM((_GE,), jnp.int32),
            pltpu.VMEM((_GE,), jnp.int32),
            pltpu.VMEM((_GE, _W), _f32),
            pltpu.VMEM((_GE, _W), _f32),
            pltpu.SemaphoreType.DMA,
            pltpu.SemaphoreType.DMA,
        ],
    )
    return f(h, src, dst)


def _scatter_body(e_hbm, d_hbm, z_hbm, agg_hbm,
                  acc, idxd, rows, c0, c1, c2, c3, c4, sem):
    cid = lax.axis_index("c")
    sid = lax.axis_index("s")
    cbase = cid * _NHALF
    chunks = (c0, c1, c2, c3, c4)

    @pl.when(sid == 0)
    def _():
        pltpu.sync_copy(z_hbm, acc)

    plsc.subcore_barrier()

    def step(i, carry):
        ebase = sid * _EPS + i * _GE
        pltpu.sync_copy(d_hbm.at[pl.ds(ebase, _GE)], idxd)
        pltpu.sync_copy(e_hbm.at[pl.ds(ebase, _GE)], rows)
        for j in range(_G):
            for v in range(_CW // 16):
                o = j * _CW + v * 16
                t = idxd[pl.ds(o, 16)] - cbase
                oob = (t < 0) | (t >= _NHALF)
                chunks[j][pl.ds(v * 16, 16)] = jnp.where(oob, _NHALF, t)
        cps = [pltpu.async_copy(rows.at[pl.ds(j * _CW, _CW)],
                                acc.at[chunks[j]], sem, add=True)
               for j in range(_G)]
        for c in cps:
            c.wait()
        return carry

    lax.fori_loop(0, _EPS // _GE, step, 0)
    plsc.subcore_barrier()

    @pl.when(sid == 0)
    def _():
        pltpu.sync_copy(acc.at[pl.ds(0, _NHALF)],
                        agg_hbm.at[pl.ds(cbase, _NHALF)])


def _scatter(e, dst, zeros):
    f = pl.kernel(
        _scatter_body,
        out_type=jax.ShapeDtypeStruct((_N, _W), _f32),
        mesh=_mesh(),
        compiler_params=pltpu.CompilerParams(use_tc_tiling_on_sc=True),
        scratch_types=[
            pltpu.VMEM_SHARED((_NACC, _W), _f32),
            pltpu.VMEM((_GE,), jnp.int32),
            pltpu.VMEM((_GE, _W), _f32),
            pltpu.VMEM((_CW,), jnp.int32),
            pltpu.VMEM((_CW,), jnp.int32),
            pltpu.VMEM((_CW,), jnp.int32),
            pltpu.VMEM((_CW,), jnp.int32),
            pltpu.VMEM((_CW,), jnp.int32),
            pltpu.SemaphoreType.DMA,
        ],
    )
    return f(e, dst, zeros)


def _dot(a, b):
    return jnp.dot(a, b, preferred_element_type=_f32)


def _node_enc_body(x_ref, w1, b1, w2, b2, o_ref):
    u = jnp.maximum(_dot(x_ref[...], w1[...]) + b1[...], 0.0)
    o_ref[...] = _dot(u, w2[...]) + b2[...]


def _node_enc(x, p):
    (W1, b1), (W2, b2) = p
    grid = _N // _BN
    cst = lambda i: (0, 0)
    return pl.pallas_call(
        _node_enc_body,
        grid=(grid,),
        in_specs=[pl.BlockSpec((_BN, _ND), lambda i: (i, 0)),
                  pl.BlockSpec((_ND, _H), cst),
                  pl.BlockSpec((1, _H), cst),
                  pl.BlockSpec((_H, _W), cst),
                  pl.BlockSpec((1, _W), cst)],
        out_specs=pl.BlockSpec((_BN, _W), lambda i: (i, 0)),
        out_shape=jax.ShapeDtypeStruct((_N, _W), _f32),
    )(x, W1, b1.reshape(1, -1), _dup_cols(W2), _dup_cols(b2.reshape(1, -1)))


def _edge_round1_body(eaT_ref, hsd_ref, e1, be1, e2, be2,
                      w1, b1, w2, b2, o_ref):
    ea = eaT_ref[...].T
    e0 = jnp.maximum(_dot(ea, e1[...]) + be1[...], 0.0)
    e0 = _dot(e0, e2[...]) + be2[...]
    u = _dot(jnp.concatenate([hsd_ref[...], e0], axis=1), w1[...])
    u = jnp.maximum(u + b1[...], 0.0)
    o_ref[...] = _dot(u, w2[...]) + b2[...]


def _edge_round1(edge_attr, hsd, enc_p, upd_p):
    (E1, be1), (E2, be2) = enc_p
    (W1, b1), (W2, b2) = upd_p
    grid = _E // _BE
    cst = lambda i: (0, 0)
    return pl.pallas_call(
        _edge_round1_body,
        grid=(grid,),
        in_specs=[pl.BlockSpec((_ED, _BE), lambda i: (0, i)),
                  pl.BlockSpec((_BE, _W), lambda i: (i, 0)),
                  pl.BlockSpec((_ED, _H), cst),
                  pl.BlockSpec((1, _H), cst),
                  pl.BlockSpec((_H, _H), cst),
                  pl.BlockSpec((1, _H), cst),
                  pl.BlockSpec((3 * _H, _W), cst),
                  pl.BlockSpec((1, _W), cst),
                  pl.BlockSpec((_W, _W), cst),
                  pl.BlockSpec((1, _W), cst)],
        out_specs=pl.BlockSpec((_BE, _W), lambda i: (i, 0)),
        out_shape=jax.ShapeDtypeStruct((_E, _W), _f32),
    )(edge_attr, hsd, E1, be1.reshape(1, -1), E2, be2.reshape(1, -1),
      W1, b1.reshape(1, -1), _dup_cols(W2), _dup_cols(b2.reshape(1, -1)))


def _edge_upd_body(e_ref, hsd_ref, w1, b1, w2, b2, o_ref):
    ein = jnp.concatenate([hsd_ref[...], e_ref[:, :_H]], axis=1)
    u = _dot(ein, w1[...])
    u = jnp.maximum(u + b1[...], 0.0)
    o_ref[...] = _dot(u, w2[...]) + b2[...]


def _edge_upd(e, hsd, upd_p):
    (W1, b1), (W2, b2) = upd_p
    grid = _E // _BE
    cst = lambda i: (0, 0)
    return pl.pallas_call(
        _edge_upd_body,
        grid=(grid,),
        in_specs=[pl.BlockSpec((_BE, _W), lambda i: (i, 0)),
                  pl.BlockSpec((_BE, _W), lambda i: (i, 0)),
                  pl.BlockSpec((3 * _H, _W), cst),
                  pl.BlockSpec((1, _W), cst),
                  pl.BlockSpec((_W, _W), cst),
                  pl.BlockSpec((1, _W), cst)],
        out_specs=pl.BlockSpec((_BE, _W), lambda i: (i, 0)),
        out_shape=jax.ShapeDtypeStruct((_E, _W), _f32),
    )(e, hsd, W1, b1.reshape(1, -1),
      _dup_cols(W2), _dup_cols(b2.reshape(1, -1)))


def _node_upd_body(h_ref, a_ref, v1, b1, v2, b2, o_ref):
    hin = jnp.concatenate([h_ref[:, :_H], a_ref[:, :_H]], axis=1)
    u = _dot(hin, v1[...])
    u = jnp.maximum(u + b1[...], 0.0)
    o_ref[...] = _dot(u, v2[...]) + b2[...]


def _node_upd(h, agg, upd_p):
    (V1, b1), (V2, b2) = upd_p
    grid = _N // _BN
    cst = lambda i: (0, 0)
    return pl.pallas_call(
        _node_upd_body,
        grid=(grid,),
        in_specs=[pl.BlockSpec((_BN, _W), lambda i: (i, 0)),
                  pl.BlockSpec((_BN, _W), lambda i: (i, 0)),
                  pl.BlockSpec((_W, _W), cst),
                  pl.BlockSpec((1, _W), cst),
                  pl.BlockSpec((_W, _W), cst),
                  pl.BlockSpec((1, _W), cst)],
        out_specs=pl.BlockSpec((_BN, _W), lambda i: (i, 0)),
        out_shape=jax.ShapeDtypeStruct((_N, _W), _f32),
    )(h, agg, V1, b1.reshape(1, -1),
      _dup_cols(V2), _dup_cols(b2.reshape(1, -1)))


def _head_body(e_ref, hsd_ref, w1, b1, w2, b2, w3, b3, o_ref):
    ein = jnp.concatenate([hsd_ref[...], e_ref[:, :_H]], axis=1)
    u = _dot(ein, w1[...])
    u = jnp.maximum(u + b1[...], 0.0)
    u = jnp.maximum(_dot(u, w2[...]) + b2[...], 0.0)
    o_ref[...] = _dot(u, w3[...]) + b3[...]


def _head(e, hsd, head_p):
    (W1, b1), (W2, b2), (W3, b3) = head_p
    grid = _E // _BE
    cst = lambda i: (0, 0)
    out = pl.pallas_call(
        _head_body,
        grid=(grid,),
        in_specs=[pl.BlockSpec((_BE, _W), lambda i: (i, 0)),
                  pl.BlockSpec((_BE, _W), lambda i: (i, 0)),
                  pl.BlockSpec((3 * _H, _W), cst),
                  pl.BlockSpec((1, _W), cst),
                  pl.BlockSpec((_W, _H), cst),
                  pl.BlockSpec((1, _H), cst),
                  pl.BlockSpec((_H, 1), cst),
                  pl.BlockSpec((1, 1), cst)],
        out_specs=pl.BlockSpec((_BE, 1), lambda i: (i, 0)),
        out_shape=jax.ShapeDtypeStruct((_E, 1), _f32),
    )(e, hsd, W1, b1.reshape(1, -1),
      W2, b2.reshape(1, -1), W3, b3.reshape(1, 1))
    return out.reshape(_E)


def kernel(x, edge_attr, params, edge_index):
    src = edge_index[0]
    dst = edge_index[1]
    zeros = jnp.zeros((_NACC, _W), _f32)

    h = _node_enc(x, params['node_enc'])

    hsd = _gather2(h, src, dst)
    e = _edge_round1(edge_attr.T, hsd, params['edge_enc'],
                     params['edge_upd'][0])
    agg = _scatter(e, dst, zeros)
    h = _node_upd(h, agg, params['node_upd'][0])

    for k in (1, 2):
        hsd = _gather2(h, src, dst)
        e = _edge_upd(e, hsd, params['edge_upd'][k])
        agg = _scatter(e, dst, zeros)
        h = _node_upd(h, agg, params['node_upd'][k])

    hsd = _gather2(h, src, dst)
    return _head(e, hsd, params['edge_head'])
